# Initial kernel scaffold; baseline (speedup 1.0000x reference)
import jax
import jax.numpy as jnp
from jax.experimental import pallas as pl


def kernel(x, W, b):
    raise NotImplementedError("write your pallas kernel here")


# speedup vs baseline: 133.9429x; 133.9429x over previous
import functools

import numpy as np
import jax
import jax.numpy as jnp
from jax import lax
from jax.experimental import pallas as pl
from jax.experimental.pallas import tpu as pltpu
from jax.experimental.pallas import tpu_sc as plsc

_DIM = 96
_K = 9
_WIN = 2 * _K + 1
_N = 112 * 112
_B = 2
_RB = 448
_NBLK = _N // _RB
_NP = _N + 2 * _RB
_NW = 32
_CHUNK = _N // _NW
_OFF = 16
_HALO = _CHUNK + 32
_G = _DIM // 16


def _dinv_np():
    n = np.arange(_N)
    deg = np.minimum(n, _K) + np.minimum(_N - 1 - n, _K) + 1
    return (1.0 / np.sqrt(deg)).astype(np.float32)


_DINV16 = np.repeat(_dinv_np()[:, None], 16, axis=1)


def _lin_body(x_ref, w_ref, o_ref):
    j = pl.program_id(1)
    rows = (j - 1) * _RB + lax.broadcasted_iota(jnp.int32, (_RB, 1), 0)
    deg = jnp.minimum(rows, _K) + jnp.minimum((_N - 1) - rows, _K) + 1
    dinv = lax.rsqrt(jnp.maximum(deg, 1).astype(jnp.float32))
    xb = x_ref[0] * dinv
    yb = jnp.dot(xb, w_ref[...], preferred_element_type=jnp.float32)
    valid = jnp.logical_and(j >= 1, j <= _NBLK)
    o_ref[0] = jnp.where(valid, yb, 0.0)


def _linear_stage(xf, w):
    return pl.pallas_call(
        _lin_body,
        grid=(_B, _NBLK + 2),
        in_specs=[
            pl.BlockSpec((1, _RB, _DIM),
                         lambda bi, j: (bi, jnp.clip(j - 1, 0, _NBLK - 1), 0)),
            pl.BlockSpec((_DIM, _DIM), lambda bi, j: (0, 0)),
        ],
        out_specs=pl.BlockSpec((1, _RB, _DIM), lambda bi, j: (bi, j, 0)),
        out_shape=jax.ShapeDtypeStruct((_B, _NP, _DIM), jnp.float32),
    )(xf, w)


def _agg_body(y_ref, dinv_ref, b_ref, out_ref, ybuf, dbuf, bbuf, obuf):
    wid = lax.axis_index("s") * 2 + lax.axis_index("c")
    st = wid * _CHUNK
    pltpu.sync_copy(b_ref, bbuf)
    pltpu.sync_copy(dinv_ref.at[pl.ds(st * 16, _CHUNK * 16)], dbuf)
    bias = [bbuf[pl.ds(g * 16, 16)] for g in range(_G)]
    lo = _OFF - _K
    for bi in range(_B):
        pltpu.sync_copy(
            y_ref.at[pl.ds((bi * _NP + _RB + st - _OFF) * _DIM, _HALO * _DIM)],
            ybuf)
        z = [ybuf[pl.ds(lo * _DIM + g * 16, 16)] for g in range(_G)]
        for r in range(lo + 1, lo + _WIN):
            for g in range(_G):
                z[g] = z[g] + ybuf[pl.ds(r * _DIM + g * 16, 16)]

        def body(i, z):
            dv = dbuf[pl.ds(i * 16, 16)]
            for g in range(_G):
                obuf[pl.ds(i * _DIM + g * 16, 16)] = z[g] * dv + bias[g]
            add = (i + lo + _WIN) * _DIM
            sub = (i + lo) * _DIM
            return tuple(
                z[g] + ybuf[pl.ds(add + g * 16, 16)]
                - ybuf[pl.ds(sub + g * 16, 16)]
                for g in range(_G))

        lax.fori_loop(0, _CHUNK, body, tuple(z))
        pltpu.sync_copy(obuf,
                        out_ref.at[pl.ds((bi * _N + st) * _DIM, _CHUNK * _DIM)])


@functools.lru_cache(maxsize=1)
def _agg_stage():
    return functools.partial(
        pl.kernel,
        out_type=jax.ShapeDtypeStruct((_B * _N * _DIM,), jnp.float32),
        mesh=plsc.VectorSubcoreMesh(core_axis_name="c", subcore_axis_name="s"),
        scratch_types=[
            pltpu.VMEM((_HALO * _DIM,), jnp.float32),
            pltpu.VMEM((_CHUNK * 16,), jnp.float32),
            pltpu.VMEM((_DIM,), jnp.float32),
            pltpu.VMEM((_CHUNK * _DIM,), jnp.float32),
        ],
    )(_agg_body)


def kernel(x, W, b):
    B, C, H, Wd = x.shape
    N = H * Wd
    x1 = x.reshape(B, C, N).transpose(0, 2, 1).reshape(B, C, H, Wd)
    xf = x1.reshape(B, C, N).transpose(0, 2, 1)
    y_pad = _linear_stage(xf, W)
    out_flat = _agg_stage()(y_pad.reshape(-1),
                            jnp.asarray(_DINV16.reshape(-1)), b)
    out_nc = out_flat.reshape(B, N, C)
    return out_nc.transpose(0, 2, 1).reshape(B, C, H, Wd)


# speedup vs baseline: 151.4064x; 1.1304x over previous
import functools

import numpy as np
import jax
import jax.numpy as jnp
from jax import lax
from jax.experimental import pallas as pl
from jax.experimental.pallas import tpu as pltpu
from jax.experimental.pallas import tpu_sc as plsc

_DIM = 96
_K = 9
_WIN = 2 * _K + 1
_N = 112 * 112
_B = 2
_RB = 896
_NBLK = _N // _RB
_NP = _N + 2 * _RB
_NW = 32
_CHUNK = _N // _NW
_OFF = 16
_HALO = _CHUNK + 32
_G = _DIM // 16


def _dinv_np():
    n = np.arange(_N)
    deg = np.minimum(n, _K) + np.minimum(_N - 1 - n, _K) + 1
    return (1.0 / np.sqrt(deg)).astype(np.float32)


_DINV16 = np.repeat(_dinv_np()[:, None], 16, axis=1)


def _lin_body(x_ref, w_ref, o_ref):
    j = pl.program_id(1)
    rows = (j - 1) * _RB + lax.broadcasted_iota(jnp.int32, (_RB, 1), 0)
    deg = jnp.minimum(rows, _K) + jnp.minimum((_N - 1) - rows, _K) + 1
    dinv = lax.rsqrt(jnp.maximum(deg, 1).astype(jnp.float32))
    yb = lax.dot_general(x_ref[0], w_ref[...],
                         dimension_numbers=(((0,), (0,)), ((), ())),
                         preferred_element_type=jnp.float32)
    valid = jnp.logical_and(j >= 1, j <= _NBLK)
    o_ref[0] = jnp.where(valid, yb * dinv, 0.0)


def _linear_stage(x_cn, w):
    return pl.pallas_call(
        _lin_body,
        grid=(_B, _NBLK + 2),
        in_specs=[
            pl.BlockSpec((1, _DIM, _RB),
                         lambda bi, j: (bi, 0, jnp.clip(j - 1, 0, _NBLK - 1))),
            pl.BlockSpec((_DIM, _DIM), lambda bi, j: (0, 0)),
        ],
        out_specs=pl.BlockSpec((1, _RB, _DIM), lambda bi, j: (bi, j, 0)),
        out_shape=jax.ShapeDtypeStruct((_B, _NP, _DIM), jnp.float32),
    )(x_cn, w)


def _agg_body(y_ref, dinv_ref, b_ref, out_ref, ybuf, dbuf, bbuf, obuf):
    wid = lax.axis_index("s") * 2 + lax.axis_index("c")
    st = wid * _CHUNK
    pltpu.sync_copy(b_ref, bbuf)
    pltpu.sync_copy(dinv_ref.at[pl.ds(st * 16, _CHUNK * 16)], dbuf)
    bias = [bbuf[pl.ds(g * 16, 16)] for g in range(_G)]
    lo = _OFF - _K
    for bi in range(_B):
        pltpu.sync_copy(
            y_ref.at[pl.ds((bi * _NP + _RB + st - _OFF) * _DIM, _HALO * _DIM)],
            ybuf)
        z = [ybuf[pl.ds(lo * _DIM + g * 16, 16)] for g in range(_G)]
        for r in range(lo + 1, lo + _WIN):
            for g in range(_G):
                z[g] = z[g] + ybuf[pl.ds(r * _DIM + g * 16, 16)]

        def body(i, z):
            dv = dbuf[pl.ds(i * 16, 16)]
            for g in range(_G):
                obuf[pl.ds(i * _DIM + g * 16, 16)] = z[g] * dv + bias[g]
            add = (i + lo + _WIN) * _DIM
            sub = (i + lo) * _DIM
            return tuple(
                z[g] + ybuf[pl.ds(add + g * 16, 16)]
                - ybuf[pl.ds(sub + g * 16, 16)]
                for g in range(_G))

        lax.fori_loop(0, _CHUNK, body, tuple(z))
        pltpu.sync_copy(obuf,
                        out_ref.at[pl.ds((bi * _N + st) * _DIM, _CHUNK * _DIM)])


@functools.lru_cache(maxsize=1)
def _agg_stage():
    return functools.partial(
        pl.kernel,
        out_type=jax.ShapeDtypeStruct((_B * _N * _DIM,), jnp.float32),
        mesh=plsc.VectorSubcoreMesh(core_axis_name="c", subcore_axis_name="s"),
        scratch_types=[
            pltpu.VMEM((_HALO * _DIM,), jnp.float32),
            pltpu.VMEM((_CHUNK * 16,), jnp.float32),
            pltpu.VMEM((_DIM,), jnp.float32),
            pltpu.VMEM((_CHUNK * _DIM,), jnp.float32),
        ],
    )(_agg_body)


def kernel(x, W, b):
    B, C, H, Wd = x.shape
    N = H * Wd
    x1 = x.reshape(B, C, N).transpose(0, 2, 1).reshape(B, C, N)
    y_pad = _linear_stage(x1, W)
    out_flat = _agg_stage()(y_pad.reshape(-1),
                            jnp.asarray(_DINV16.reshape(-1)), b)
    out_nc = out_flat.reshape(B, N, C)
    return out_nc.transpose(0, 2, 1).reshape(B, C, H, Wd)


# speedup vs baseline: 191.6590x; 1.2659x over previous
import functools

import numpy as np
import jax
import jax.numpy as jnp
from jax import lax
from jax.experimental import pallas as pl
from jax.experimental.pallas import tpu as pltpu
from jax.experimental.pallas import tpu_sc as plsc

_DIM = 96
_K = 9
_WIN = 2 * _K + 1
_N = 112 * 112
_B = 2
_RB = 896
_NBLK = _N // _RB
_NP = _N + 2 * _RB
_NW = 32
_CHUNK = _N // _NW
_OFF = 16
_HALO = _CHUNK + 32
_G = _DIM // 16


def _dinv_np():
    n = np.arange(_N)
    deg = np.minimum(n, _K) + np.minimum(_N - 1 - n, _K) + 1
    return (1.0 / np.sqrt(deg)).astype(np.float32)


_DINV16 = np.repeat(_dinv_np()[:, None], 16, axis=1)


def _lin_body(x_ref, w_ref, o_ref):
    j = pl.program_id(1)
    rows = (j - 1) * _RB + lax.broadcasted_iota(jnp.int32, (_RB, 1), 0)
    deg = jnp.minimum(rows, _K) + jnp.minimum((_N - 1) - rows, _K) + 1
    dinv = lax.rsqrt(jnp.maximum(deg, 1).astype(jnp.float32))
    yb = lax.dot_general(x_ref[0], w_ref[...],
                         dimension_numbers=(((0,), (0,)), ((), ())),
                         preferred_element_type=jnp.float32)
    valid = jnp.logical_and(j >= 1, j <= _NBLK)
    o_ref[0] = jnp.where(valid, yb * dinv, 0.0)


def _linear_stage(x_cn, w):
    return pl.pallas_call(
        _lin_body,
        grid=(_B, _NBLK + 2),
        in_specs=[
            pl.BlockSpec((1, _DIM, _RB),
                         lambda bi, j: (bi, 0, jnp.clip(j - 1, 0, _NBLK - 1))),
            pl.BlockSpec((_DIM, _DIM), lambda bi, j: (0, 0)),
        ],
        out_specs=pl.BlockSpec((1, _RB, _DIM), lambda bi, j: (bi, j, 0)),
        out_shape=jax.ShapeDtypeStruct((_B, _NP, _DIM), jnp.float32),
    )(x_cn, w)


def _agg_body(y_ref, dinv_ref, b_ref, out_ref, ybuf, dbuf, bbuf, obuf):
    wid = lax.axis_index("s") * 2 + lax.axis_index("c")
    st = wid * _CHUNK
    pltpu.sync_copy(b_ref, bbuf)
    pltpu.sync_copy(dinv_ref.at[pl.ds(st * 16, _CHUNK * 16)], dbuf)
    bias = [bbuf[pl.ds(g * 16, 16)] for g in range(_G)]
    lo = _OFF - _K
    for bi in range(_B):
        pltpu.sync_copy(y_ref.at[bi, pl.ds(_RB + st - _OFF, _HALO), :], ybuf)
        z = [ybuf[lo, pl.ds(g * 16, 16)] for g in range(_G)]
        for r in range(lo + 1, lo + _WIN):
            for g in range(_G):
                z[g] = z[g] + ybuf[r, pl.ds(g * 16, 16)]

        def body(i, z):
            dv = dbuf[pl.ds(i * 16, 16)]
            for g in range(_G):
                obuf[i, pl.ds(g * 16, 16)] = z[g] * dv + bias[g]
            return tuple(
                z[g] + ybuf[i + lo + _WIN, pl.ds(g * 16, 16)]
                - ybuf[i + lo, pl.ds(g * 16, 16)]
                for g in range(_G))

        lax.fori_loop(0, _CHUNK, body, tuple(z))
        pltpu.sync_copy(obuf, out_ref.at[bi, pl.ds(st, _CHUNK), :])


@functools.lru_cache(maxsize=1)
def _agg_stage():
    return functools.partial(
        pl.kernel,
        out_type=jax.ShapeDtypeStruct((_B, _N, _DIM), jnp.float32),
        mesh=plsc.VectorSubcoreMesh(core_axis_name="c", subcore_axis_name="s"),
        scratch_types=[
            pltpu.VMEM((_HALO, _DIM), jnp.float32),
            pltpu.VMEM((_CHUNK * 16,), jnp.float32),
            pltpu.VMEM((_DIM,), jnp.float32),
            pltpu.VMEM((_CHUNK, _DIM), jnp.float32),
        ],
    )(_agg_body)


def kernel(x, W, b):
    B, C, H, Wd = x.shape
    N = H * Wd
    x1 = x.reshape(B, C, N).transpose(0, 2, 1).reshape(B, C, N)
    y_pad = _linear_stage(x1, W)
    out_nc = _agg_stage()(y_pad, jnp.asarray(_DINV16.reshape(-1)), b)
    return out_nc.transpose(0, 2, 1).reshape(B, C, H, Wd)
